# baseline (speedup 1.0000x reference)
import functools

import jax
import jax.numpy as jnp
from jax import lax
from jax.experimental import pallas as pl
from jax.experimental.pallas import tpu as pltpu
from jax.experimental.pallas import tpu_sc as plsc

D = 128
TAB = 64
ZERO_ROW = 52

NC = 2
NS = 16
NW = NC * NS

CH = 128


def _table_body(rank_ref, suit_ref, out_ref):
    cid = lax.broadcasted_iota(jnp.int32, (TAB, D), 0)
    col = lax.broadcasted_iota(jnp.int32, (TAB, D), 1)
    oh_r = (cid % 13 == col).astype(jnp.float32)
    oh_s = (cid // 13 == col).astype(jnp.float32)
    tab = (jnp.dot(oh_r, rank_ref[...], preferred_element_type=jnp.float32,
                   precision=lax.Precision.HIGHEST)
           + jnp.dot(oh_s, suit_ref[...], preferred_element_type=jnp.float32,
                     precision=lax.Precision.HIGHEST))
    out_ref[...] = jnp.where(cid < ZERO_ROW, tab, 0.0)


def _build_table(rank_pad, suit_pad):
    return pl.pallas_call(
        _table_body,
        out_shape=jax.ShapeDtypeStruct((TAB, D), jnp.float32),
    )(rank_pad, suit_pad)


def _make_sc_gather(n_tokens):
    rows_w = n_tokens // NW
    nch = rows_w // CH
    mesh = plsc.VectorSubcoreMesh(core_axis_name="c", subcore_axis_name="s")

    @functools.partial(
        pl.kernel,
        mesh=mesh,
        out_type=jax.ShapeDtypeStruct((n_tokens, D), jnp.float32),
        scratch_types=[
            pltpu.VMEM((nch, CH), jnp.int32),
            pltpu.VMEM((nch, CH), jnp.int32),
            pltpu.VMEM((CH, D), jnp.float32),
            pltpu.SemaphoreType.DMA,
        ],
    )
    def sc_gather(table_hbm, ids_hbm, mask_hbm, out_hbm, ids_v, idx_v, buf, gsem):
        wid = lax.axis_index("s") * NC + lax.axis_index("c")
        blk = wid * nch
        pltpu.sync_copy(ids_hbm.at[pl.ds(blk, nch)], ids_v)
        pltpu.sync_copy(mask_hbm.at[pl.ds(blk, nch)], idx_v)

        def cbody(r, carry):
            for j in range(CH // 16):
                sl = pl.ds(j * 16, 16)
                m = idx_v[r, sl]
                v = ids_v[r, sl]
                idx_v[r, sl] = ZERO_ROW + m * (v - ZERO_ROW)
            return carry

        lax.fori_loop(0, nch, cbody, 0)

        base = wid * rows_w

        def gbody(g, carry):
            pltpu.async_copy(table_hbm.at[idx_v.at[g]], buf, gsem).wait()
            pltpu.sync_copy(buf, out_hbm.at[pl.ds(base + g * CH, CH)])
            return carry

        lax.fori_loop(0, nch, gbody, 0)

    return sc_gather


def kernel(card_ids, slot_mask, rank_emb, suit_emb):
    b, l = card_ids.shape
    n = b * l
    ids2 = card_ids.astype(jnp.int32).reshape(n // CH, CH)
    mask2 = slot_mask.astype(jnp.int32).reshape(n // CH, CH)

    rank_pad = jnp.zeros((D, D), jnp.float32).at[:13].set(rank_emb)
    suit_pad = jnp.zeros((D, D), jnp.float32).at[:4].set(suit_emb)
    table = _build_table(rank_pad, suit_pad)

    toks = _make_sc_gather(n)(table, ids2, mask2)
    return toks.reshape(b, l, D), slot_mask.astype(bool)


# speedup vs baseline: 10.9371x; 10.9371x over previous
import functools

import jax
import jax.numpy as jnp
from jax import lax
from jax.experimental import pallas as pl
from jax.experimental.pallas import tpu as pltpu
from jax.experimental.pallas import tpu_sc as plsc

D = 128
TAB = 64
ZERO_ROW = 52

NC = 2
NS = 16
NW = NC * NS

RPC = 4


def _prep_body(ids_ref, msk_ref, rank_ref, suit_ref, idx_ref, tab_ref):
    rows, l = ids_ref.shape
    ids = ids_ref[...]
    m = msk_ref[...]
    sel = ZERO_ROW + m * (ids - ZERO_ROW)
    pad = jnp.full((rows, D - l), ZERO_ROW, jnp.int32)
    idx_ref[...] = jnp.concatenate([sel, pad], axis=1)

    cid = lax.broadcasted_iota(jnp.int32, (TAB, D), 0)
    col = lax.broadcasted_iota(jnp.int32, (TAB, D), 1)
    oh_r = (cid % 13 == col).astype(jnp.float32)
    oh_s = (cid // 13 == col).astype(jnp.float32)
    tab = (jnp.dot(oh_r, rank_ref[...], preferred_element_type=jnp.float32,
                   precision=lax.Precision.HIGHEST)
           + jnp.dot(oh_s, suit_ref[...], preferred_element_type=jnp.float32,
                     precision=lax.Precision.HIGHEST))
    tab_ref[...] = jnp.where(cid < ZERO_ROW, tab, 0.0)


def _prep(card_ids, slot_mask, rank_pad, suit_pad):
    b, l = card_ids.shape
    rows = b // NW
    return pl.pallas_call(
        _prep_body,
        grid=(NW,),
        in_specs=[
            pl.BlockSpec((rows, l), lambda i: (i, 0)),
            pl.BlockSpec((rows, l), lambda i: (i, 0)),
            pl.BlockSpec((D, D), lambda i: (0, 0)),
            pl.BlockSpec((D, D), lambda i: (0, 0)),
        ],
        out_specs=[
            pl.BlockSpec((rows, D), lambda i: (i, 0)),
            pl.BlockSpec((TAB, D), lambda i: (0, 0)),
        ],
        out_shape=[
            jax.ShapeDtypeStruct((b, D), jnp.int32),
            jax.ShapeDtypeStruct((TAB, D), jnp.float32),
        ],
    )(card_ids, slot_mask, rank_pad, suit_pad)


def _make_sc_lookup(b, l):
    rows_w = b // NW
    nch = rows_w // RPC
    cl = RPC * l
    full16 = l // 16
    rem = l - full16 * 16
    mesh = plsc.VectorSubcoreMesh(core_axis_name="c", subcore_axis_name="s")

    @functools.partial(
        pl.kernel,
        mesh=mesh,
        out_type=jax.ShapeDtypeStruct((b * l, D), jnp.float32),
        scratch_types=[
            pltpu.VMEM((TAB, D), jnp.float32),
            pltpu.VMEM((rows_w, D), jnp.int32),
            pltpu.VMEM((cl, D), jnp.float32),
            pltpu.VMEM((cl, D), jnp.float32),
            pltpu.SemaphoreType.DMA,
            pltpu.SemaphoreType.DMA,
        ],
    )
    def sc_lookup(tab_hbm, idx_hbm, out_hbm, tab_v, idx_v, buf0, buf1, sem0, sem1):
        wid = lax.axis_index("s") * NC + lax.axis_index("c")
        pltpu.sync_copy(tab_hbm, tab_v)
        pltpu.sync_copy(idx_hbm.at[pl.ds(wid * rows_w, rows_w)], idx_v)
        base = wid * rows_w * l

        def fill(buf, bslot, tvec, nk):
            for k in range(nk):
                t = tvec[k]
                for j in range(D // 16):
                    sl = pl.ds(j * 16, 16)
                    buf[bslot + k, sl] = tab_v[t, sl]

        def pair(h, carry):
            for bb, buf, sem in ((0, buf0, sem0), (1, buf1, sem1)):
                c = 2 * h + bb

                @pl.when(h > 0)
                def _drain():
                    pltpu.make_async_copy(buf, out_hbm.at[pl.ds(0, cl)], sem).wait()

                def rbody(rr, cr):
                    row = c * RPC + rr
                    bslot = rr * l
                    for g in range(full16):
                        fill(buf, bslot + g * 16, idx_v[row, pl.ds(g * 16, 16)], 16)
                    if rem:
                        fill(buf, bslot + full16 * 16,
                             idx_v[row, pl.ds(full16 * 16, 16)], rem)
                    return cr

                lax.fori_loop(0, RPC, rbody, 0)
                pltpu.async_copy(buf, out_hbm.at[pl.ds(base + c * cl, cl)], sem)
            return carry

        lax.fori_loop(0, nch // 2, pair, 0)
        pltpu.make_async_copy(buf0, out_hbm.at[pl.ds(0, cl)], sem0).wait()
        pltpu.make_async_copy(buf1, out_hbm.at[pl.ds(0, cl)], sem1).wait()

    return sc_lookup


def kernel(card_ids, slot_mask, rank_emb, suit_emb):
    b, l = card_ids.shape
    rank_pad = jnp.zeros((D, D), jnp.float32).at[:13].set(rank_emb)
    suit_pad = jnp.zeros((D, D), jnp.float32).at[:4].set(suit_emb)
    idx_pad, table = _prep(card_ids.astype(jnp.int32), slot_mask.astype(jnp.int32),
                           rank_pad, suit_pad)
    toks = _make_sc_lookup(b, l)(table, idx_pad)
    return toks.reshape(b, l, D), slot_mask.astype(bool)


# speedup vs baseline: 21.6583x; 1.9803x over previous
import functools

import jax
import jax.numpy as jnp
from jax import lax
from jax.experimental import pallas as pl
from jax.experimental.pallas import tpu as pltpu
from jax.experimental.pallas import tpu_sc as plsc

D = 128
TAB = 64
ZERO_ROW = 52

NC = 2
NS = 16
NW = NC * NS

CB = 128


def _prep_body(ids_ref, msk_ref, rank_ref, suit_ref, idx_ref, tab_ref):
    l, cols = ids_ref.shape
    lp = idx_ref.shape[0]
    ids = ids_ref[...]
    m = msk_ref[...]
    sel = ZERO_ROW + m * (ids - ZERO_ROW)
    pad = jnp.full((lp - l, cols), ZERO_ROW, jnp.int32)
    idx_ref[...] = jnp.concatenate([sel, pad], axis=0)

    cid = lax.broadcasted_iota(jnp.int32, (TAB, D), 0)
    col = lax.broadcasted_iota(jnp.int32, (TAB, D), 1)
    oh_r = (cid % 13 == col).astype(jnp.float32)
    oh_s = (cid // 13 == col).astype(jnp.float32)
    tab = (jnp.dot(oh_r, rank_ref[...], preferred_element_type=jnp.float32,
                   precision=lax.Precision.HIGHEST)
           + jnp.dot(oh_s, suit_ref[...], preferred_element_type=jnp.float32,
                     precision=lax.Precision.HIGHEST))
    tab_ref[...] = jnp.where(cid < ZERO_ROW, tab, 0.0)


def _prep(ids_t, msk_t, rank_pad, suit_pad):
    l, b = ids_t.shape
    lp = (l + 7) // 8 * 8
    cols = b // NW
    return pl.pallas_call(
        _prep_body,
        grid=(NW,),
        in_specs=[
            pl.BlockSpec((l, cols), lambda i: (0, i)),
            pl.BlockSpec((l, cols), lambda i: (0, i)),
            pl.BlockSpec((D, D), lambda i: (0, 0)),
            pl.BlockSpec((D, D), lambda i: (0, 0)),
        ],
        out_specs=[
            pl.BlockSpec((lp, cols), lambda i: (0, i)),
            pl.BlockSpec((TAB, D), lambda i: (0, 0)),
        ],
        out_shape=[
            jax.ShapeDtypeStruct((lp, b), jnp.int32),
            jax.ShapeDtypeStruct((TAB, D), jnp.float32),
        ],
    )(ids_t, msk_t, rank_pad, suit_pad)


def _make_sc_lookup(b, l, lp):
    bw = b // NW
    ncb = bw // CB
    mesh = plsc.VectorSubcoreMesh(core_axis_name="c", subcore_axis_name="s")

    @functools.partial(
        pl.kernel,
        mesh=mesh,
        out_type=jax.ShapeDtypeStruct((l, b, D), jnp.float32),
        scratch_types=[
            pltpu.VMEM((TAB, D), jnp.float32),
            pltpu.VMEM((lp, bw), jnp.int32),
            pltpu.VMEM((CB, D), jnp.float32),
            pltpu.VMEM((CB, D), jnp.float32),
            pltpu.SemaphoreType.DMA,
            pltpu.SemaphoreType.DMA,
        ],
    )
    def sc_lookup(tab_hbm, idx_hbm, out_hbm, tab_v, idx_v, buf0, buf1, sem0, sem1):
        wid = lax.axis_index("s") * NC + lax.axis_index("c")
        b0 = wid * bw
        pltpu.sync_copy(tab_hbm, tab_v)
        pltpu.sync_copy(idx_hbm.at[pl.ds(0, lp), pl.ds(b0, bw)], idx_v)

        def lbody(li, carry):
            for c in range(ncb):
                buf = buf0 if c % 2 == 0 else buf1
                sem = sem0 if c % 2 == 0 else sem1

                if c < 2:
                    @pl.when(li > 0)
                    def _drain():
                        pltpu.make_async_copy(
                            buf, out_hbm.at[0, pl.ds(0, CB)], sem).wait()
                else:
                    pltpu.make_async_copy(
                        buf, out_hbm.at[0, pl.ds(0, CB)], sem).wait()

                def gbody(g, cr):
                    tvec = idx_v[li, pl.ds(c * CB + g * 16, 16)]
                    for k in range(16):
                        t = tvec[k]
                        row = g * 16 + k
                        for j in range(D // 16):
                            sl = pl.ds(j * 16, 16)
                            buf[row, sl] = tab_v[t, sl]
                    return cr

                lax.fori_loop(0, CB // 16, gbody, 0)
                pltpu.async_copy(buf, out_hbm.at[li, pl.ds(b0 + c * CB, CB)], sem)
            return carry

        lax.fori_loop(0, l, lbody, 0)
        pltpu.make_async_copy(buf0, out_hbm.at[0, pl.ds(0, CB)], sem0).wait()
        pltpu.make_async_copy(buf1, out_hbm.at[0, pl.ds(0, CB)], sem1).wait()

    return sc_lookup


def kernel(card_ids, slot_mask, rank_emb, suit_emb):
    b, l = card_ids.shape
    rank_pad = jnp.zeros((D, D), jnp.float32).at[:13].set(rank_emb)
    suit_pad = jnp.zeros((D, D), jnp.float32).at[:4].set(suit_emb)
    idx_t, table = _prep(card_ids.astype(jnp.int32).T, slot_mask.astype(jnp.int32).T,
                         rank_pad, suit_pad)
    lp = idx_t.shape[0]
    out_lbd = _make_sc_lookup(b, l, lp)(table, idx_t)
    return out_lbd.transpose(1, 0, 2), slot_mask.astype(bool)


# speedup vs baseline: 39.5117x; 1.8243x over previous
import functools

import jax
import jax.numpy as jnp
from jax import lax
from jax.experimental import pallas as pl
from jax.experimental.pallas import tpu as pltpu
from jax.experimental.pallas import tpu_sc as plsc

D = 128
TAB = 64
ZERO_ROW = 52

NC = 2
NS = 16
NW = NC * NS

CB = 128


def _prep_body(ids_ref, msk_ref, rank_ref, suit_ref, idx_ref, tab_ref):
    l, cols = ids_ref.shape
    lp = idx_ref.shape[0]
    ids = ids_ref[...]
    m = msk_ref[...]
    sel = ZERO_ROW + m * (ids - ZERO_ROW)
    pad = jnp.full((lp - l, cols), ZERO_ROW, jnp.int32)
    idx_ref[...] = jnp.concatenate([sel, pad], axis=0)

    cid = lax.broadcasted_iota(jnp.int32, (TAB, D), 0)
    col = lax.broadcasted_iota(jnp.int32, (TAB, D), 1)
    oh_r = (cid % 13 == col).astype(jnp.float32)
    oh_s = (cid // 13 == col).astype(jnp.float32)
    tab = (jnp.dot(oh_r, rank_ref[...], preferred_element_type=jnp.float32,
                   precision=lax.Precision.HIGHEST)
           + jnp.dot(oh_s, suit_ref[...], preferred_element_type=jnp.float32,
                     precision=lax.Precision.HIGHEST))
    tab_ref[...] = jnp.where(cid < ZERO_ROW, tab, 0.0)


def _prep(ids_t, msk_t, rank_pad, suit_pad):
    l, b = ids_t.shape
    lp = (l + 7) // 8 * 8
    cols = b // NW
    return pl.pallas_call(
        _prep_body,
        grid=(NW,),
        in_specs=[
            pl.BlockSpec((l, cols), lambda i: (0, i)),
            pl.BlockSpec((l, cols), lambda i: (0, i)),
            pl.BlockSpec((D, D), lambda i: (0, 0)),
            pl.BlockSpec((D, D), lambda i: (0, 0)),
        ],
        out_specs=[
            pl.BlockSpec((lp, cols), lambda i: (0, i)),
            pl.BlockSpec((TAB, D), lambda i: (0, 0)),
        ],
        out_shape=[
            jax.ShapeDtypeStruct((lp, b), jnp.int32),
            jax.ShapeDtypeStruct((TAB, D), jnp.float32),
        ],
    )(ids_t, msk_t, rank_pad, suit_pad)


def _make_sc_lookup(b, l, lp):
    bw = b // NW
    ncb = bw // CB
    mesh = plsc.VectorSubcoreMesh(core_axis_name="c", subcore_axis_name="s")

    @functools.partial(
        pl.kernel,
        mesh=mesh,
        out_type=jax.ShapeDtypeStruct((l, b, D), jnp.float32),
        scratch_types=[
            pltpu.VMEM((TAB, D), jnp.float32),
            pltpu.VMEM((lp, bw), jnp.int32),
            pltpu.VMEM((CB, D), jnp.float32),
            pltpu.VMEM((CB, D), jnp.float32),
            pltpu.SemaphoreType.DMA,
            pltpu.SemaphoreType.DMA,
        ],
    )
    def sc_lookup(tab_hbm, idx_hbm, out_hbm, tab_v, idx_v, buf0, buf1, sem0, sem1):
        wid = lax.axis_index("s") * NC + lax.axis_index("c")
        b0 = wid * bw
        pltpu.sync_copy(tab_hbm, tab_v)
        pltpu.sync_copy(idx_hbm.at[pl.ds(0, lp), pl.ds(b0, bw)], idx_v)

        def lbody(li, carry):
            for c in range(ncb):
                buf = buf0 if c % 2 == 0 else buf1
                sem = sem0 if c % 2 == 0 else sem1

                if c < 2:
                    @pl.when(li > 0)
                    def _drain():
                        pltpu.make_async_copy(
                            buf, out_hbm.at[0, pl.ds(0, CB)], sem).wait()
                else:
                    pltpu.make_async_copy(
                        buf, out_hbm.at[0, pl.ds(0, CB)], sem).wait()

                @plsc.parallel_loop(0, CB // 16, 1, unroll=2)
                def _gbody(g):
                    tvec = idx_v[li, pl.ds(c * CB + g * 16, 16)]
                    for k in range(16):
                        t = tvec[k]
                        row = g * 16 + k
                        for j in range(D // 16):
                            sl = pl.ds(j * 16, 16)
                            buf[row, sl] = tab_v[t, sl]
                pltpu.async_copy(buf, out_hbm.at[li, pl.ds(b0 + c * CB, CB)], sem)
            return carry

        lax.fori_loop(0, l, lbody, 0)
        pltpu.make_async_copy(buf0, out_hbm.at[0, pl.ds(0, CB)], sem0).wait()
        pltpu.make_async_copy(buf1, out_hbm.at[0, pl.ds(0, CB)], sem1).wait()

    return sc_lookup


def kernel(card_ids, slot_mask, rank_emb, suit_emb):
    b, l = card_ids.shape
    rank_pad = jnp.zeros((D, D), jnp.float32).at[:13].set(rank_emb)
    suit_pad = jnp.zeros((D, D), jnp.float32).at[:4].set(suit_emb)
    idx_t, table = _prep(card_ids.astype(jnp.int32).T, slot_mask.astype(jnp.int32).T,
                         rank_pad, suit_pad)
    lp = idx_t.shape[0]
    out_lbd = _make_sc_lookup(b, l, lp)(table, idx_t)
    return out_lbd.transpose(1, 0, 2), slot_mask.astype(bool)
